# baseline (speedup 1.0000x reference)
import functools

import jax
import jax.numpy as jnp
from jax import lax
from jax.experimental import pallas as pl
from jax.experimental.pallas import tpu as pltpu
from jax.experimental.pallas import tpu_sc as plsc

_B = 128
_FG = 30000
_N = _B * _FG
_NW = 32
_PER_W = _N // _NW
_CHUNK = 12000
_NCHUNK = _PER_W // _CHUNK
_GROUPS = _CHUNK // 48

_TINY = 1.1754943508222875e-38
_LN2 = 0.6931471805599453
_SQRT2 = 1.4142135


def _ln(v):
    i = lax.bitcast_convert_type(v, jnp.int32)
    e = lax.shift_right_logical(i, 23) - 127
    m = lax.bitcast_convert_type(
        jnp.bitwise_or(jnp.bitwise_and(i, 0x007FFFFF), 0x3F800000), jnp.float32)
    big = m > _SQRT2
    m = jnp.where(big, m * jnp.float32(0.5), m)
    ef = (e + big.astype(jnp.int32)).astype(jnp.float32)
    s = (m - jnp.float32(1.0)) / (m + jnp.float32(1.0))
    z = s * s
    p = jnp.float32(1.0 / 9.0)
    for c in (1.0 / 7.0, 1.0 / 5.0, 1.0 / 3.0, 1.0):
        p = p * z + jnp.float32(c)
    return ef * jnp.float32(_LN2) + jnp.float32(2.0) * s * p


_MESH = plsc.VectorSubcoreMesh(core_axis_name="c", subcore_axis_name="s")


@functools.partial(
    pl.kernel,
    mesh=_MESH,
    compiler_params=pltpu.CompilerParams(needs_layout_passes=False),
    out_type=jax.ShapeDtypeStruct((_N,), jnp.float32),
    scratch_types=[
        pltpu.VMEM((_CHUNK,), jnp.float32),
        pltpu.VMEM((_CHUNK,), jnp.float32),
        pltpu.VMEM((_CHUNK,), jnp.float32),
    ],
)
def _sc_gumbel_softmax(x_hbm, u_hbm, out_hbm, xb, ub, ob):
    wid = lax.axis_index("s") * 2 + lax.axis_index("c")
    base = wid * _PER_W
    iota3 = lax.iota(jnp.int32, 16) * 3

    def chunk_body(ci, carry):
        off = base + ci * _CHUNK
        pltpu.sync_copy(x_hbm.at[pl.ds(off, _CHUNK)], xb)
        pltpu.sync_copy(u_hbm.at[pl.ds(off, _CHUNK)], ub)

        def grp(k, carry2):
            i0 = iota3 + k * 48
            i1 = i0 + 1
            i2 = i0 + 2
            x0 = plsc.load_gather(xb, [i0])
            x1 = plsc.load_gather(xb, [i1])
            x2 = plsc.load_gather(xb, [i2])
            u0 = plsc.load_gather(ub, [i0])
            u1 = plsc.load_gather(ub, [i1])
            u2 = plsc.load_gather(ub, [i2])
            xm = jnp.maximum(x0, jnp.maximum(x1, x2))
            tiny = jnp.float32(_TINY)
            l0 = _ln(jnp.maximum(u0, tiny))
            l1 = _ln(jnp.maximum(u1, tiny))
            l2 = _ln(jnp.maximum(u2, tiny))
            two = jnp.float32(2.0)
            w0 = jnp.exp((x0 - xm) * two) / (l0 * l0)
            w1 = jnp.exp((x1 - xm) * two) / (l1 * l1)
            w2 = jnp.exp((x2 - xm) * two) / (l2 * l2)
            r = jnp.float32(1.0) / (w0 + w1 + w2)
            plsc.store_scatter(ob, [i0], w0 * r)
            plsc.store_scatter(ob, [i1], w1 * r)
            plsc.store_scatter(ob, [i2], w2 * r)
            return carry2

        lax.fori_loop(0, _GROUPS, grp, 0, unroll=False)
        pltpu.sync_copy(ob, out_hbm.at[pl.ds(off, _CHUNK)])
        return carry

    lax.fori_loop(0, _NCHUNK, chunk_body, 0, unroll=False)


def kernel(logits, uniform):
    x = logits.reshape(_N)
    u = uniform.reshape(_N)
    out = _sc_gumbel_softmax(x, u)
    return out.reshape(logits.shape)


# speedup vs baseline: 42.3292x; 42.3292x over previous
import functools

import jax
import jax.numpy as jnp
from jax import lax
from jax.experimental import pallas as pl
from jax.experimental.pallas import tpu as pltpu
from jax.experimental.pallas import tpu_sc as plsc

_B = 128
_F = 10000
_N = _B * 3 * _F
_NW = 32
_PF = 313
_CF = 32
_NCHUNK = 10
_BLK = 3 * _B
_CHUNK = _CF * _BLK

_TINY = 1.1754943508222875e-38
_LN2 = 0.6931471805599453
_SQRT2 = 1.4142135


def _ln(v):
    i = lax.bitcast_convert_type(v, jnp.int32)
    e = lax.shift_right_logical(i, 23) - 127
    m = lax.bitcast_convert_type(
        jnp.bitwise_or(jnp.bitwise_and(i, 0x007FFFFF), 0x3F800000), jnp.float32)
    big = m > _SQRT2
    m = jnp.where(big, m * jnp.float32(0.5), m)
    ef = (e + big.astype(jnp.int32)).astype(jnp.float32)
    s = (m - jnp.float32(1.0)) / (m + jnp.float32(1.0))
    z = s * s
    p = jnp.float32(1.0 / 9.0)
    for c in (1.0 / 7.0, 1.0 / 5.0, 1.0 / 3.0, 1.0):
        p = p * z + jnp.float32(c)
    return ef * jnp.float32(_LN2) + jnp.float32(2.0) * s * p


_MESH = plsc.VectorSubcoreMesh(core_axis_name="c", subcore_axis_name="s")


@functools.partial(
    pl.kernel,
    mesh=_MESH,
    compiler_params=pltpu.CompilerParams(needs_layout_passes=False),
    out_type=jax.ShapeDtypeStruct((_N,), jnp.float32),
    scratch_types=[
        pltpu.VMEM((_CHUNK,), jnp.float32),
        pltpu.VMEM((_CHUNK,), jnp.float32),
        pltpu.VMEM((_CHUNK,), jnp.float32),
    ],
)
def _sc_gumbel_softmax(x_hbm, u_hbm, out_hbm, xb, ub, ob):
    wid = lax.axis_index("s") * 2 + lax.axis_index("c")
    f0 = jnp.minimum(wid * _PF, _F - _PF)

    def chunk_body(ci, carry):
        fi = f0 + jnp.minimum(ci * _CF, _PF - _CF)
        off = fi * _BLK
        pltpu.sync_copy(x_hbm.at[pl.ds(off, _CHUNK)], xb)
        pltpu.sync_copy(u_hbm.at[pl.ds(off, _CHUNK)], ub)

        def vec(j, carry2):
            o0 = lax.shift_right_logical(j, 3) * _BLK + jnp.bitwise_and(j, 7) * 16
            o1 = o0 + _B
            o2 = o1 + _B
            x0 = xb[pl.ds(o0, 16)]
            x1 = xb[pl.ds(o1, 16)]
            x2 = xb[pl.ds(o2, 16)]
            u0 = ub[pl.ds(o0, 16)]
            u1 = ub[pl.ds(o1, 16)]
            u2 = ub[pl.ds(o2, 16)]
            xm = jnp.maximum(x0, jnp.maximum(x1, x2))
            tiny = jnp.float32(_TINY)
            l0 = _ln(jnp.maximum(u0, tiny))
            l1 = _ln(jnp.maximum(u1, tiny))
            l2 = _ln(jnp.maximum(u2, tiny))
            a = l0 * l0
            b = l1 * l1
            c = l2 * l2
            two = jnp.float32(2.0)
            e0 = jnp.exp((x0 - xm) * two) * (b * c)
            e1 = jnp.exp((x1 - xm) * two) * (a * c)
            e2 = jnp.exp((x2 - xm) * two) * (a * b)
            r = jnp.float32(1.0) / (e0 + e1 + e2)
            ob[pl.ds(o0, 16)] = e0 * r
            ob[pl.ds(o1, 16)] = e1 * r
            ob[pl.ds(o2, 16)] = e2 * r
            return carry2

        lax.fori_loop(0, _CF * 8, vec, 0, unroll=False)
        pltpu.sync_copy(ob, out_hbm.at[pl.ds(off, _CHUNK)])
        return carry

    lax.fori_loop(0, _NCHUNK, chunk_body, 0, unroll=False)


def kernel(logits, uniform):
    x = logits.T.reshape(_N)
    u = uniform.transpose(1, 2, 3, 0).reshape(_N)
    out = _sc_gumbel_softmax(x, u)
    return out.reshape(3 * _F, _B).T
